# baseline (speedup 1.0000x reference)
import functools
import math

import jax
import jax.numpy as jnp
from jax import lax
from jax.experimental import pallas as pl
from jax.experimental.pallas import tpu as pltpu
from jax.experimental.pallas import tpu_sc as plsc

VOCAB = 1_000_000
EMB = 32
BATCH = 4096
SEQ = 200

_info = plsc.get_sparse_core_info()
NC = _info.num_cores
NS = _info.num_subcores
NW = NC * NS
BB = 512
NBLK = SEQ * (BATCH // BB)
PER_W = NBLK // NW
QB = BATCH // BB
SCALE = math.sqrt(EMB)

_mesh = plsc.VectorSubcoreMesh(core_axis_name="c", subcore_axis_name="s")


@functools.partial(
    pl.kernel,
    out_type=jax.ShapeDtypeStruct((SEQ, EMB // 8, BATCH // 128, 8, 128), jnp.float32),
    mesh=_mesh,
    scratch_types=[
        pltpu.VMEM((BB,), jnp.int32),
        pltpu.VMEM((BB,), jnp.int32),
        pltpu.VMEM((BB, EMB), jnp.float32),
        pltpu.VMEM((BB, EMB), jnp.float32),
        pltpu.VMEM((EMB // 8, BB // 128, 8, 128), jnp.float32),
        pltpu.VMEM((EMB // 8, BB // 128, 8, 128), jnp.float32),
        pltpu.SemaphoreType.DMA,
        pltpu.SemaphoreType.DMA,
        pltpu.SemaphoreType.DMA,
        pltpu.SemaphoreType.DMA,
        pltpu.SemaphoreType.DMA,
        pltpu.SemaphoreType.DMA,
    ],
    compiler_params=pltpu.CompilerParams(
        use_tc_tiling_on_sc=False, needs_layout_passes=False
    ),
)
def _embed_sc(tok_hbm, table_hbm, out_hbm,
              idx0, idx1, rows0, rows1, t0, t1,
              i0, i1, g0, g1, w0, w1):
    wid = lax.axis_index("s") * NC + lax.axis_index("c")
    idx = (idx0, idx1)
    rows = (rows0, rows1)
    tb_ = (t0, t1)
    isem = (i0, i1)
    gsem = (g0, g1)
    wsem = (w0, w1)
    lanes = jax.lax.iota(jnp.int32, 16)

    def addr(i):
        blk = wid * PER_W + i
        s = blk // QB
        tb0 = (blk % QB) * (BB // 128)
        return s, s // 8, s % 8, tb0

    def idx_copies(i, b):
        s, ts, si, tb0 = addr(i)
        return [
            pltpu.make_async_copy(
                tok_hbm.at[ts, tb0 + k, si], idx[b].at[pl.ds(k * 128, 128)], isem[b]
            )
            for k in range(BB // 128)
        ]

    def gather(i, b):
        return pltpu.make_async_copy(table_hbm.at[idx[b]], rows[b], gsem[b])

    def wback(i, b):
        s, ts, si, tb0 = addr(i)
        return pltpu.make_async_copy(
            tb_[b], out_hbm.at[s, :, pl.ds(tb0, BB // 128), :, :], wsem[b]
        )

    for c in idx_copies(0, 0):
        c.start()
    for c in idx_copies(0, 0):
        c.wait()
    gather(0, 0).start()
    for c in idx_copies(1, 1):
        c.start()

    for i in range(PER_W):
        b = i & 1
        if i + 1 < PER_W:
            for c in idx_copies(i + 1, 1 - b):
                c.wait()
            gather(i + 1, 1 - b).start()
        if i + 2 < PER_W:
            for c in idx_copies(i + 2, b):
                c.start()
        gather(i, b).wait()
        if i >= 2:
            wback(i - 2, b).wait()

        rows_b = rows[b]
        t_b = tb_[b]

        def transpose_scale(j, carry):
            te = j >> 8
            r = j & 255
            tq = r >> 6
            r2 = r & 63
            ei = r2 >> 3
            bg = r2 & 7
            col = te * 8 + ei
            row_idx = tq * 128 + bg * 16 + lanes
            col_idx = jnp.full((16,), 0, jnp.int32) + col
            v = plsc.load_gather(rows_b, [row_idx, col_idx]) * SCALE
            t_b[te, tq, ei, pl.ds(bg * 16, 16)] = v
            return carry

        lax.fori_loop(0, (EMB // 8) * (BB // 128) * 8 * 8, transpose_scale, 0,
                      unroll=4)
        wback(i, b).start()

    wback(PER_W - 2, PER_W & 1).wait()
    wback(PER_W - 1, 1 - (PER_W & 1)).wait()


def kernel(tokens, embedding_weight):
    tokq = tokens.reshape(BATCH // 128, 128, SEQ // 8, 8).transpose(2, 0, 3, 1)
    w = _embed_sc(tokq, embedding_weight)
    return w.transpose(2, 4, 0, 1, 3).reshape(BATCH, SEQ, EMB)


# speedup vs baseline: 1.1735x; 1.1735x over previous
import functools
import math

import jax
import jax.numpy as jnp
from jax import lax
from jax.experimental import pallas as pl
from jax.experimental.pallas import tpu as pltpu
from jax.experimental.pallas import tpu_sc as plsc

VOCAB = 1_000_000
EMB = 32
BATCH = 4096
SEQ = 200

_info = plsc.get_sparse_core_info()
NC = _info.num_cores
NS = _info.num_subcores
NW = NC * NS
BB = 512
NBLK = SEQ * (BATCH // BB)
PER_W = NBLK // NW
QB = BATCH // BB
SCALE = math.sqrt(EMB)

_mesh = plsc.VectorSubcoreMesh(core_axis_name="c", subcore_axis_name="s")


@functools.partial(
    pl.kernel,
    out_type=jax.ShapeDtypeStruct((SEQ, EMB // 8, BATCH // 128, 8, 128), jnp.float32),
    mesh=_mesh,
    scratch_types=[
        pltpu.VMEM((BB,), jnp.int32),
        pltpu.VMEM((BB,), jnp.int32),
        pltpu.VMEM((BB, EMB), jnp.float32),
        pltpu.VMEM((BB, EMB), jnp.float32),
        pltpu.VMEM((EMB // 8, BB // 128, 8, 128), jnp.float32),
        pltpu.VMEM((EMB // 8, BB // 128, 8, 128), jnp.float32),
        pltpu.SemaphoreType.DMA,
        pltpu.SemaphoreType.DMA,
        pltpu.SemaphoreType.DMA,
        pltpu.SemaphoreType.DMA,
        pltpu.SemaphoreType.DMA,
        pltpu.SemaphoreType.DMA,
    ],
    compiler_params=pltpu.CompilerParams(
        use_tc_tiling_on_sc=False, needs_layout_passes=False
    ),
)
def _embed_sc(tok_hbm, table_hbm, out_hbm,
              idx0, idx1, rows0, rows1, t0, t1,
              i0, i1, g0, g1, w0, w1):
    wid = lax.axis_index("s") * NC + lax.axis_index("c")
    idx = (idx0, idx1)
    rows = (rows0, rows1)
    tb_ = (t0, t1)
    isem = (i0, i1)
    gsem = (g0, g1)
    wsem = (w0, w1)
    lanes = jax.lax.iota(jnp.int32, 16)

    def addr(i):
        blk = wid * PER_W + i
        s = blk // QB
        tb0 = (blk % QB) * (BB // 128)
        return s, s // 8, s % 8, tb0

    def idx_copies(i, b):
        s, ts, si, tb0 = addr(i)
        return [
            pltpu.make_async_copy(
                tok_hbm.at[ts, tb0 + k, si], idx[b].at[pl.ds(k * 128, 128)], isem[b]
            )
            for k in range(BB // 128)
        ]

    def gather(i, b):
        return pltpu.make_async_copy(table_hbm.at[idx[b]], rows[b], gsem[b])

    def wback(i, b):
        s, ts, si, tb0 = addr(i)
        return pltpu.make_async_copy(
            tb_[b], out_hbm.at[s, :, pl.ds(tb0, BB // 128), :, :], wsem[b]
        )

    for c in idx_copies(0, 0):
        c.start()
    for c in idx_copies(0, 0):
        c.wait()
    gather(0, 0).start()
    for c in idx_copies(1, 1):
        c.start()

    for i in range(PER_W):
        b = i & 1
        if i + 1 < PER_W:
            for c in idx_copies(i + 1, 1 - b):
                c.wait()
            gather(i + 1, 1 - b).start()
        if i + 2 < PER_W:
            for c in idx_copies(i + 2, b):
                c.start()
        gather(i, b).wait()
        if i >= 2:
            wback(i - 2, b).wait()

        rows_b = rows[b]
        t_b = tb_[b]

        @plsc.parallel_loop(0, (EMB // 8) * (BB // 128) * 8 * 8, unroll=8)
        def transpose_scale(j):
            te = j >> 8
            r = j & 255
            tq = r >> 6
            r2 = r & 63
            ei = r2 >> 3
            bg = r2 & 7
            col = te * 8 + ei
            row_idx = tq * 128 + bg * 16 + lanes
            col_idx = jnp.full((16,), 0, jnp.int32) + col
            v = plsc.load_gather(rows_b, [row_idx, col_idx]) * SCALE
            t_b[te, tq, ei, pl.ds(bg * 16, 16)] = v
        wback(i, b).start()

    wback(PER_W - 2, PER_W & 1).wait()
    wback(PER_W - 1, 1 - (PER_W & 1)).wait()


def kernel(tokens, embedding_weight):
    tokq = tokens.reshape(BATCH // 128, 128, SEQ // 8, 8).transpose(2, 0, 3, 1)
    w = _embed_sc(tokq, embedding_weight)
    return w.transpose(2, 4, 0, 1, 3).reshape(BATCH, SEQ, EMB)


# speedup vs baseline: 1.2801x; 1.0909x over previous
import functools
import math

import jax
import jax.numpy as jnp
from jax import lax
from jax.experimental import pallas as pl
from jax.experimental.pallas import tpu as pltpu
from jax.experimental.pallas import tpu_sc as plsc

VOCAB = 1_000_000
EMB = 32
BATCH = 4096
SEQ = 200

_info = plsc.get_sparse_core_info()
NC = _info.num_cores
NS = _info.num_subcores
NW = NC * NS
BB = 512
NBLK = SEQ * (BATCH // BB)
PER_W = NBLK // NW
QB = BATCH // BB
TQ = BB // 128
TSZ = TQ * 8 * 128
SCALE = math.sqrt(EMB)

_mesh = plsc.VectorSubcoreMesh(core_axis_name="c", subcore_axis_name="s")


@functools.partial(
    pl.kernel,
    out_type=jax.ShapeDtypeStruct((SEQ, EMB // 8, (BATCH // 128) * 1024), jnp.float32),
    mesh=_mesh,
    scratch_types=[
        pltpu.VMEM((BB,), jnp.int32),
        pltpu.VMEM((BB,), jnp.int32),
        pltpu.VMEM((BB, EMB), jnp.float32),
        pltpu.VMEM((BB, EMB), jnp.float32),
        pltpu.VMEM((EMB // 8 * TSZ,), jnp.float32),
        pltpu.VMEM((EMB // 8 * TSZ,), jnp.float32),
        pltpu.SemaphoreType.DMA,
        pltpu.SemaphoreType.DMA,
        pltpu.SemaphoreType.DMA,
        pltpu.SemaphoreType.DMA,
        pltpu.SemaphoreType.DMA,
        pltpu.SemaphoreType.DMA,
    ],
    compiler_params=pltpu.CompilerParams(
        use_tc_tiling_on_sc=False, needs_layout_passes=False
    ),
)
def _embed_sc(tok_hbm, table_hbm, out_hbm,
              idx0, idx1, rows0, rows1, t0, t1,
              i0, i1, g0, g1, w0, w1):
    wid = lax.axis_index("s") * NC + lax.axis_index("c")
    idx = (idx0, idx1)
    rows = (rows0, rows1)
    tb_ = (t0, t1)
    isem = (i0, i1)
    gsem = (g0, g1)
    wsem = (w0, w1)
    lanes = jax.lax.iota(jnp.int32, 16)
    tbase0 = (lanes >> 3) * TSZ + (lanes & 7) * 128
    tbase1 = tbase0 + 2 * TSZ

    def addr(i):
        blk = wid * PER_W + i
        s = blk // QB
        tb0 = (blk % QB) * TQ
        return s, s // 8, s % 8, tb0

    def idx_copies(i, b):
        s, ts, si, tb0 = addr(i)
        return [
            pltpu.make_async_copy(
                tok_hbm.at[ts, tb0 + k, si], idx[b].at[pl.ds(k * 128, 128)], isem[b]
            )
            for k in range(TQ)
        ]

    def gather(i, b):
        return pltpu.make_async_copy(table_hbm.at[idx[b]], rows[b], gsem[b])

    def wbacks(i, b):
        s, ts, si, tb0 = addr(i)
        return [
            pltpu.make_async_copy(
                tb_[b].at[pl.ds(te * TSZ, TSZ)],
                out_hbm.at[s, te, pl.ds(tb0 * 1024, TSZ)],
                wsem[b],
            )
            for te in range(EMB // 8)
        ]

    for c in idx_copies(0, 0):
        c.start()
    for c in idx_copies(0, 0):
        c.wait()
    gather(0, 0).start()
    for c in idx_copies(1, 1):
        c.start()

    for i in range(PER_W):
        b = i & 1
        if i + 1 < PER_W:
            for c in idx_copies(i + 1, 1 - b):
                c.wait()
            gather(i + 1, 1 - b).start()
        if i + 2 < PER_W:
            for c in idx_copies(i + 2, b):
                c.start()
        gather(i, b).wait()
        if i >= 2:
            for c in wbacks(i - 2, b):
                c.wait()

        rows_b = rows[b]
        t_b = tb_[b]

        @plsc.parallel_loop(0, BB, unroll=8)
        def transpose_scale(r):
            base = (r >> 7) * 1024 + (r & 127)
            v0 = rows_b[r, pl.ds(0, 16)] * SCALE
            v1 = rows_b[r, pl.ds(16, 16)] * SCALE
            plsc.store_scatter(t_b, [tbase0 + base], v0)
            plsc.store_scatter(t_b, [tbase1 + base], v1)

        for c in wbacks(i, b):
            c.start()

    for c in wbacks(PER_W - 2, PER_W & 1):
        c.wait()
    for c in wbacks(PER_W - 1, 1 - (PER_W & 1)):
        c.wait()


def kernel(tokens, embedding_weight):
    tokq = tokens.reshape(BATCH // 128, 128, SEQ // 8, 8).transpose(2, 0, 3, 1)
    w = _embed_sc(tokq, embedding_weight)
    w = w.reshape(SEQ, EMB // 8, BATCH // 128, 8, 128)
    return w.transpose(2, 4, 0, 1, 3).reshape(BATCH, SEQ, EMB)


# speedup vs baseline: 1.3488x; 1.0537x over previous
import functools
import math

import jax
import jax.numpy as jnp
from jax import lax
from jax.experimental import pallas as pl
from jax.experimental.pallas import tpu as pltpu
from jax.experimental.pallas import tpu_sc as plsc

VOCAB = 1_000_000
EMB = 32
BATCH = 4096
SEQ = 200

_info = plsc.get_sparse_core_info()
NC = _info.num_cores
NS = _info.num_subcores
NW = NC * NS
BB = 512
NBLK = SEQ * (BATCH // BB)
PER_W = NBLK // NW
QB = BATCH // BB
TQ = BB // 128
NS_IDX = 7
SCALE = math.sqrt(EMB)

_mesh = plsc.VectorSubcoreMesh(core_axis_name="c", subcore_axis_name="s")


@functools.partial(
    pl.kernel,
    out_type=jax.ShapeDtypeStruct((SEQ, EMB // 8, BATCH // 128, 1024), jnp.float32),
    mesh=_mesh,
    scratch_types=[
        pltpu.VMEM((NS_IDX, BATCH), jnp.int32),
        pltpu.VMEM((BB, EMB), jnp.float32),
        pltpu.VMEM((BB, EMB), jnp.float32),
        pltpu.VMEM((EMB // 8, TQ, 1024), jnp.float32),
        pltpu.VMEM((EMB // 8, TQ, 1024), jnp.float32),
        pltpu.SemaphoreType.DMA,
        pltpu.SemaphoreType.DMA,
        pltpu.SemaphoreType.DMA,
        pltpu.SemaphoreType.DMA,
        pltpu.SemaphoreType.DMA,
    ],
    compiler_params=pltpu.CompilerParams(
        use_tc_tiling_on_sc=False, needs_layout_passes=False
    ),
)
def _embed_sc(tok_hbm, table_hbm, out_hbm,
              idx_all, rows0, rows1, t0, t1,
              i0, g0, g1, w0, w1):
    wid = lax.axis_index("s") * NC + lax.axis_index("c")
    rows = (rows0, rows1)
    tb_ = (t0, t1)
    gsem = (g0, g1)
    wsem = (w0, w1)
    s_lo = (wid * PER_W) // QB
    lanes = jax.lax.iota(jnp.int32, 16)
    te_lo = lanes >> 3
    te_hi = te_lo + 2
    ei_off = (lanes & 7) * 128

    def addr(i):
        blk = wid * PER_W + i
        s = blk // QB
        tb0 = (blk % QB) * TQ
        return s, tb0

    def gather(i, b):
        s, tb0 = addr(i)
        return pltpu.make_async_copy(
            table_hbm.at[idx_all.at[s - s_lo, pl.ds(tb0 * 128, BB)]],
            rows[b], gsem[b],
        )

    def wback(i, b):
        s, tb0 = addr(i)
        return pltpu.make_async_copy(
            tb_[b], out_hbm.at[s, :, pl.ds(tb0, TQ), :], wsem[b]
        )

    pltpu.make_async_copy(
        tok_hbm.at[pl.ds(s_lo, NS_IDX)], idx_all, i0
    ).start()
    pltpu.make_async_copy(
        tok_hbm.at[pl.ds(s_lo, NS_IDX)], idx_all, i0
    ).wait()
    gather(0, 0).start()

    for i in range(PER_W):
        b = i & 1
        if i + 1 < PER_W:
            gather(i + 1, 1 - b).start()
        gather(i, b).wait()
        if i >= 2:
            wback(i - 2, b).wait()

        rows_b = rows[b]
        t_b = tb_[b]

        @plsc.parallel_loop(0, BB, unroll=8)
        def transpose_scale(r):
            tq = r >> 7
            bi = r & 127
            tqv = jnp.full((16,), 0, jnp.int32) + tq
            eib = ei_off + bi
            v0 = rows_b[r, pl.ds(0, 16)] * SCALE
            v1 = rows_b[r, pl.ds(16, 16)] * SCALE
            plsc.store_scatter(t_b, [te_lo, tqv, eib], v0)
            plsc.store_scatter(t_b, [te_hi, tqv, eib], v1)

        wback(i, b).start()

    wback(PER_W - 2, PER_W & 1).wait()
    wback(PER_W - 1, 1 - (PER_W & 1)).wait()


def kernel(tokens, embedding_weight):
    w = _embed_sc(tokens.T, embedding_weight)
    w = w.reshape(SEQ, EMB // 8, BATCH // 128, 8, 128)
    return w.transpose(2, 4, 0, 1, 3).reshape(BATCH, SEQ, EMB)


# speedup vs baseline: 1.9608x; 1.4537x over previous
import functools
import math

import jax
import jax.numpy as jnp
from jax import lax
from jax.experimental import pallas as pl
from jax.experimental.pallas import tpu as pltpu
from jax.experimental.pallas import tpu_sc as plsc

VOCAB = 1_000_000
EMB = 32
BATCH = 4096
SEQ = 200

_info = plsc.get_sparse_core_info()
NC = _info.num_cores
NS = _info.num_subcores
NW = NC * NS
BB = 512
NBLK = SEQ * (BATCH // BB)
PER_W = NBLK // NW
QB = BATCH // BB
TQ = BB // 128
NS_IDX = 7
SCALE = math.sqrt(EMB)

_mesh = plsc.VectorSubcoreMesh(core_axis_name="c", subcore_axis_name="s")


@functools.partial(
    pl.kernel,
    out_type=jax.ShapeDtypeStruct((SEQ, EMB // 8, BATCH // 128, 8, 128), jnp.float32),
    mesh=_mesh,
    scratch_types=[
        pltpu.VMEM((NS_IDX, BATCH), jnp.int32),
        pltpu.VMEM((BB, EMB), jnp.float32),
        pltpu.VMEM((BB, EMB), jnp.float32),
        pltpu.VMEM((EMB // 8, TQ, 8, 129), jnp.float32),
        pltpu.VMEM((EMB // 8, TQ, 8, 129), jnp.float32),
        pltpu.SemaphoreType.DMA,
        pltpu.SemaphoreType.DMA,
        pltpu.SemaphoreType.DMA,
        pltpu.SemaphoreType.DMA,
        pltpu.SemaphoreType.DMA,
    ],
    compiler_params=pltpu.CompilerParams(
        use_tc_tiling_on_sc=False, needs_layout_passes=False
    ),
)
def _embed_sc(tok_hbm, table_hbm, out_hbm,
              idx_all, rows0, rows1, t0, t1,
              i0, g0, g1, w0, w1):
    wid = lax.axis_index("s") * NC + lax.axis_index("c")
    rows = (rows0, rows1)
    tb_ = (t0, t1)
    gsem = (g0, g1)
    wsem = (w0, w1)
    s_lo = (wid * PER_W) // QB
    lanes = jax.lax.iota(jnp.int32, 16)
    te_lo = lanes >> 3
    te_hi = te_lo + 2
    ei_l = lanes & 7

    def addr(i):
        blk = wid * PER_W + i
        s = blk // QB
        tb0 = (blk % QB) * TQ
        return s, tb0

    def gather(i, b):
        s, tb0 = addr(i)
        return pltpu.make_async_copy(
            table_hbm.at[idx_all.at[s - s_lo, pl.ds(tb0 * 128, BB)]],
            rows[b], gsem[b],
        )

    def wback(i, b):
        s, tb0 = addr(i)
        return pltpu.make_async_copy(
            tb_[b].at[:, :, :, pl.ds(0, 128)],
            out_hbm.at[s, :, pl.ds(tb0, TQ), :, :],
            wsem[b],
        )

    pltpu.make_async_copy(
        tok_hbm.at[pl.ds(s_lo, NS_IDX)], idx_all, i0
    ).start()
    pltpu.make_async_copy(
        tok_hbm.at[pl.ds(s_lo, NS_IDX)], idx_all, i0
    ).wait()
    gather(0, 0).start()

    for i in range(PER_W):
        b = i & 1
        if i + 1 < PER_W:
            gather(i + 1, 1 - b).start()
        gather(i, b).wait()
        if i >= 2:
            wback(i - 2, b).wait()

        rows_b = rows[b]
        t_b = tb_[b]

        @plsc.parallel_loop(0, BB, unroll=8)
        def transpose_scale(r):
            tq = r >> 7
            bi = r & 127
            tqv = jnp.full((16,), 0, jnp.int32) + tq
            biv = jnp.full((16,), 0, jnp.int32) + bi
            v0 = rows_b[r, pl.ds(0, 16)] * SCALE
            v1 = rows_b[r, pl.ds(16, 16)] * SCALE
            plsc.store_scatter(t_b, [te_lo, tqv, ei_l, biv], v0)
            plsc.store_scatter(t_b, [te_hi, tqv, ei_l, biv], v1)

        wback(i, b).start()

    wback(PER_W - 2, PER_W & 1).wait()
    wback(PER_W - 1, 1 - (PER_W & 1)).wait()


def kernel(tokens, embedding_weight):
    w = _embed_sc(tokens.T, embedding_weight)
    return w.transpose(2, 4, 0, 1, 3).reshape(BATCH, SEQ, EMB)


# speedup vs baseline: 1.9796x; 1.0096x over previous
import functools
import math

import jax
import jax.numpy as jnp
from jax import lax
from jax.experimental import pallas as pl
from jax.experimental.pallas import tpu as pltpu
from jax.experimental.pallas import tpu_sc as plsc

VOCAB = 1_000_000
EMB = 32
BATCH = 4096
SEQ = 200

_info = plsc.get_sparse_core_info()
NC = _info.num_cores
NS = _info.num_subcores
NW = NC * NS
BB = 512
NBLK = 1600
PER_W = NBLK // NW
SCALE = math.sqrt(EMB)

_mesh = plsc.VectorSubcoreMesh(core_axis_name="c", subcore_axis_name="s")


@functools.partial(
    pl.kernel,
    out_type=jax.ShapeDtypeStruct((SEQ, EMB // 8, BATCH // 128, 8, 128), jnp.float32),
    mesh=_mesh,
    scratch_types=[
        pltpu.VMEM((BB,), jnp.int32),
        pltpu.VMEM((BB,), jnp.int32),
        pltpu.VMEM((BB, EMB), jnp.float32),
        pltpu.VMEM((BB, EMB), jnp.float32),
        pltpu.VMEM((4, EMB // 8, 1, 8, 129), jnp.float32),
        pltpu.VMEM((4, EMB // 8, 1, 8, 129), jnp.float32),
        pltpu.SemaphoreType.DMA,
        pltpu.SemaphoreType.DMA,
        pltpu.SemaphoreType.DMA,
        pltpu.SemaphoreType.DMA,
        pltpu.SemaphoreType.DMA,
        pltpu.SemaphoreType.DMA,
    ],
    compiler_params=pltpu.CompilerParams(
        use_tc_tiling_on_sc=False, needs_layout_passes=False
    ),
)
def _embed_sc(tok_hbm, table_hbm, out_hbm,
              idx0, idx1, rows0, rows1, t0, t1,
              i0, i1, g0, g1, w0, w1):
    wid = lax.axis_index("s") * NC + lax.axis_index("c")
    idx = (idx0, idx1)
    isem = (i0, i1)
    rows = (rows0, rows1)
    tb_ = (t0, t1)
    gsem = (g0, g1)
    wsem = (w0, w1)
    lanes = jax.lax.iota(jnp.int32, 16)
    te_lo = lanes >> 3
    te_hi = te_lo + 2
    ei_l = lanes & 7
    zerov = jnp.full((16,), 0, jnp.int32)

    def addr(i):
        blk = wid * PER_W + i
        ts = blk // 64
        rem = blk % 64
        return ts, rem // 32, rem % 32

    def idx_copy(i, b):
        ts, h, tb = addr(i)
        return pltpu.make_async_copy(
            tok_hbm.at[ts, tb, pl.ds(512 * h, BB)], idx[b], isem[b]
        )

    def gather(i, b):
        return pltpu.make_async_copy(table_hbm.at[idx[b]], rows[b], gsem[b])

    def wback(i, b):
        ts, h, tb = addr(i)
        return pltpu.make_async_copy(
            tb_[b].at[:, :, :, :, pl.ds(0, 128)],
            out_hbm.at[pl.ds(ts * 8 + 4 * h, 4), :, pl.ds(tb, 1), :, :],
            wsem[b],
        )

    idx_copy(0, 0).start()
    idx_copy(0, 0).wait()
    gather(0, 0).start()
    idx_copy(1, 1).start()

    for i in range(PER_W):
        b = i & 1
        if i + 1 < PER_W:
            idx_copy(i + 1, 1 - b).wait()
            gather(i + 1, 1 - b).start()
        if i + 2 < PER_W:
            idx_copy(i + 2, b).start()
        gather(i, b).wait()
        if i >= 2:
            wback(i - 2, b).wait()

        rows_b = rows[b]
        t_b = tb_[b]

        @plsc.parallel_loop(0, BB, unroll=8)
        def transpose_scale(r):
            sj = r >> 7
            bi = r & 127
            sjv = zerov + sj
            biv = zerov + bi
            v0 = rows_b[r, pl.ds(0, 16)] * SCALE
            v1 = rows_b[r, pl.ds(16, 16)] * SCALE
            plsc.store_scatter(t_b, [sjv, te_lo, zerov, ei_l, biv], v0)
            plsc.store_scatter(t_b, [sjv, te_hi, zerov, ei_l, biv], v1)

        wback(i, b).start()

    wback(PER_W - 2, PER_W & 1).wait()
    wback(PER_W - 1, 1 - (PER_W & 1)).wait()


def kernel(tokens, embedding_weight):
    tokq = (tokens.reshape(BATCH // 128, 128, SEQ // 8, 8)
            .transpose(2, 0, 3, 1).reshape(SEQ // 8, BATCH // 128, 1024))
    w = _embed_sc(tokq, embedding_weight)
    return w.transpose(2, 4, 0, 1, 3).reshape(BATCH, SEQ, EMB)
